# Initial kernel scaffold; baseline (speedup 1.0000x reference)
import jax
import jax.numpy as jnp
from jax.experimental import pallas as pl


def kernel(pitch, amplitude, y, WT, conv_w, conv_b, duration_secs):
    raise NotImplementedError("write your pallas kernel here")


# speedup vs baseline: 8.0512x; 8.0512x over previous
import functools

import jax
import jax.numpy as jnp
from jax import lax
from jax.experimental import pallas as pl
from jax.experimental.pallas import tpu as pltpu

_SR = 44100
_WT_LEN = 512
_N_WT = 64
_TB = 2048


def _block_cumsum(x, tb):
    sh = 1
    while sh < tb:
        x = x + jnp.pad(x[:, :-sh], ((0, 0), (sh, 0)))
        sh *= 2
    return x


def _k1_body(pitch_ref, y_ref, wt_ref, att_ref, il_ref, alpha_ref, carry_ref):
    g = pl.program_id(0)
    b, tb = pitch_ref.shape

    @pl.when(g == 0)
    def _init():
        carry_ref[...] = jnp.zeros_like(carry_ref)

    inc = pitch_ref[...] * (_WT_LEN / _SR)
    cs = _block_cumsum(inc, tb)
    carry = carry_ref[:b, :1]
    index = cs + carry - inc[0:1, :]
    carry_ref[:b, :1] = carry + cs[:, -1:]

    index = jnp.mod(index, float(_WT_LEN))
    low = jnp.floor(index)
    alpha = index - low
    il = jnp.clip(low.astype(jnp.int32), 0, _WT_LEN - 1)
    ih = jnp.where(il == _WT_LEN - 1, 0, il + 1)

    iota = lax.broadcasted_iota(jnp.int32, (_WT_LEN, tb), 0)
    c1 = jnp.zeros((_WT_LEN, tb), jnp.float32)
    for bb in range(b):
        w = y_ref[bb : bb + 1, :] * (1.0 / b)
        wlo = w * (1.0 - alpha[bb : bb + 1, :])
        whi = w * alpha[bb : bb + 1, :]
        c1 = c1 + jnp.where(iota == il[bb : bb + 1, :], wlo, 0.0)
        c1 = c1 + jnp.where(iota == ih[bb : bb + 1, :], whi, 0.0)

    att_ref[...] = jnp.dot(wt_ref[...], c1, preferred_element_type=jnp.float32)
    il_ref[...] = il
    alpha_ref[...] = alpha


def _k2_body(main_ref, left_ref, right_ref, il_ref, alpha_ref, amp_ref,
             wt_ref, cw_ref, out_ref):
    g = pl.program_id(0)
    b, tb = il_ref.shape

    lh = jnp.where(g == 0, 0.0, left_ref[:, -2:])
    window = jnp.concatenate([lh, main_ref[...], right_ref[:, :2]], axis=1)
    att = jnp.full((_N_WT, tb), cw_ref[5], jnp.float32)
    for k in range(5):
        att = att + cw_ref[k] * window[:, k : k + tb]

    mx = jnp.max(att, axis=0, keepdims=True)
    e = jnp.exp(att - mx)
    s = e / jnp.sum(e, axis=0, keepdims=True)

    il = il_ref[...]
    alpha = alpha_ref[...]
    ih = jnp.where(il == _WT_LEN - 1, 0, il + 1)
    iota = lax.broadcasted_iota(jnp.int32, (_WT_LEN, tb), 0)
    rows = []
    for bb in range(b):
        m = jnp.where(iota == il[bb : bb + 1, :], 1.0 - alpha[bb : bb + 1, :], 0.0)
        m = m + jnp.where(iota == ih[bb : bb + 1, :], alpha[bb : bb + 1, :], 0.0)
        mixed = jnp.dot(wt_ref[...], m, preferred_element_type=jnp.float32)
        rows.append(jnp.sum(s * mixed, axis=0, keepdims=True))
    out_ref[...] = jnp.concatenate(rows, axis=0) * amp_ref[...]


@jax.jit
def _run(pitch, amplitude, y, WT, conv_w, conv_b):
    b, t = pitch.shape
    g = pl.cdiv(t, _TB)
    tpad = g * _TB
    pad = tpad - t
    pitch_p = jnp.pad(pitch, ((0, 0), (0, pad)))
    y_p = jnp.pad(y, ((0, 0), (0, pad)))
    amp_p = jnp.pad(amplitude[..., 0], ((0, 0), (0, pad)))
    cw = jnp.concatenate([conv_w.reshape(5), conv_b.reshape(1),
                          jnp.zeros((2,), jnp.float32)])

    att, il, alpha = pl.pallas_call(
        _k1_body,
        grid=(g,),
        in_specs=[
            pl.BlockSpec((b, _TB), lambda i: (0, i)),
            pl.BlockSpec((b, _TB), lambda i: (0, i)),
            pl.BlockSpec((_N_WT, _WT_LEN), lambda i: (0, 0)),
        ],
        out_specs=[
            pl.BlockSpec((_N_WT, _TB), lambda i: (0, i)),
            pl.BlockSpec((b, _TB), lambda i: (0, i)),
            pl.BlockSpec((b, _TB), lambda i: (0, i)),
        ],
        out_shape=[
            jax.ShapeDtypeStruct((_N_WT, tpad), jnp.float32),
            jax.ShapeDtypeStruct((b, tpad), jnp.int32),
            jax.ShapeDtypeStruct((b, tpad), jnp.float32),
        ],
        scratch_shapes=[pltpu.VMEM((8, 128), jnp.float32)],
    )(pitch_p, y_p, WT)

    nhb = _TB // 128
    last_hb = tpad // 128 - 1
    out = pl.pallas_call(
        _k2_body,
        grid=(g,),
        in_specs=[
            pl.BlockSpec((_N_WT, _TB), lambda i: (0, i)),
            pl.BlockSpec((_N_WT, 128), lambda i: (0, jnp.maximum(i * nhb - 1, 0))),
            pl.BlockSpec((_N_WT, 128), lambda i: (0, jnp.minimum((i + 1) * nhb, last_hb))),
            pl.BlockSpec((b, _TB), lambda i: (0, i)),
            pl.BlockSpec((b, _TB), lambda i: (0, i)),
            pl.BlockSpec((b, _TB), lambda i: (0, i)),
            pl.BlockSpec((_N_WT, _WT_LEN), lambda i: (0, 0)),
            pl.BlockSpec(memory_space=pltpu.SMEM),
        ],
        out_specs=pl.BlockSpec((b, _TB), lambda i: (0, i)),
        out_shape=jax.ShapeDtypeStruct((b, tpad), jnp.float32),
    )(att, att, att, il, alpha, amp_p, WT, cw)

    return out[:, :t, None]


def kernel(pitch, amplitude, y, WT, conv_w, conv_b, duration_secs):
    return _run(pitch, amplitude, y, WT, conv_w, conv_b)
